# baseline (speedup 1.0000x reference)
import functools

import numpy as np

import jax
import jax.numpy as jnp
from jax import lax
from jax.experimental import pallas as pl
from jax.experimental.pallas import tpu as pltpu
from jax.experimental.pallas import tpu_sc as plsc


@functools.lru_cache(maxsize=None)
def _make_sc_gather(B, V, D, F):
    info = plsc.get_sparse_core_info()
    NC, NS, L = info.num_cores, info.num_subcores, info.num_lanes
    NW = NC * NS
    assert F % NW == 0 and D % L == 0
    b_per_w = F // NW
    CH = 64
    assert b_per_w % CH == 0
    NCH = b_per_w // CH
    NBUF = 3
    LEAD = NBUF - 1
    mesh = plsc.VectorSubcoreMesh(core_axis_name="c", subcore_axis_name="s")

    @functools.partial(
        pl.kernel,
        mesh=mesh,
        out_type=jax.ShapeDtypeStruct((B, D), jnp.float32),
        scratch_types=[
            pltpu.VMEM((b_per_w,), jnp.float32),
            pltpu.VMEM((NCH, CH), jnp.int32),
            pltpu.VMEM((NBUF, CH, D), jnp.float32),
        ]
        + [pltpu.SemaphoreType.DMA] * (2 * NBUF),
    )
    def k(t_hbm, pe_hbm, out_hbm, t_v, idx_v, rows_v, *sems):
        gsem = sems[:NBUF]
        osem = sems[NBUF:]
        wid = lax.axis_index("s") * NC + lax.axis_index("c")
        base = wid * b_per_w
        pltpu.sync_copy(t_hbm.at[pl.ds(base, b_per_w)], t_v)
        scale = jnp.float32(V - 1)

        def compute_idx(c):
            for j in range(CH // L):
                tv = t_v[pl.ds(c * CH + j * L, L)]
                iv = jnp.clip((tv * scale).astype(jnp.int32), 0, V - 1)
                idx_v[c, pl.ds(j * L, L)] = iv

        def gather(c):
            b = c % NBUF
            return pltpu.async_copy(pe_hbm.at[idx_v.at[c]], rows_v.at[b], gsem[b])

        def put(c):
            b = c % NBUF
            return pltpu.async_copy(
                rows_v.at[b], out_hbm.at[pl.ds(base + c * CH, CH)], osem[b])

        gpend = [None] * NBUF
        opend = [None] * NBUF
        for i in range(NCH + LEAD):
            cg = i
            if cg < NCH:
                b = cg % NBUF
                if opend[b] is not None:
                    opend[b].wait()
                    opend[b] = None
                compute_idx(cg)
                gpend[b] = gather(cg)
            cp = i - LEAD
            if 0 <= cp < NCH:
                b = cp % NBUF
                gpend[b].wait()
                opend[b] = put(cp)
        for p in opend:
            if p is not None:
                p.wait()

    return k


@functools.lru_cache(maxsize=None)
def _make_tc_fill(B, V, D, F, R=512):
    assert (B - F) % R == 0
    nblk = (B - F) // R
    fblk = F // R
    assert F % R == 0
    scale = float(V - 1)
    half_pi = float(np.pi / 2)

    def body(t_ref, dd_ref, _, out_ref):
        tv = t_ref[...]
        idx = jnp.clip((tv * scale).astype(jnp.int32), 0, V - 1)
        pos = idx.astype(jnp.float32)
        arg = pos * dd_ref[...]
        lane = lax.broadcasted_iota(jnp.int32, (R, D), 1)
        shift = jnp.where((lane & 1) == 1, jnp.float32(half_pi), jnp.float32(0.0))
        out_ref[...] = jnp.sin(arg + shift)

    return pl.pallas_call(
        body,
        grid=(nblk,),
        in_specs=[
            pl.BlockSpec((R, 1), lambda i: (i + fblk, 0)),
            pl.BlockSpec((1, D), lambda i: (0, 0)),
            pl.BlockSpec((R, D), lambda i: (i + fblk, 0)),
        ],
        out_specs=pl.BlockSpec((R, D), lambda i: (i + fblk, 0)),
        out_shape=jax.ShapeDtypeStruct((B, D), jnp.float32),
        input_output_aliases={2: 0},
    )


@functools.lru_cache(maxsize=None)
def _dd_row(D):
    div = np.exp(np.arange(0, D, 2).astype(np.float32) * (-np.log(10000.0) / D))
    return jnp.asarray(np.repeat(div, 2).reshape(1, D))


_SC_FRACTION_NUM, _SC_FRACTION_DEN = 1, 2


def kernel(t, pe):
    B, = t.shape
    V, D = pe.shape
    F = (B * _SC_FRACTION_NUM // _SC_FRACTION_DEN) // 2048 * 2048
    out = _make_sc_gather(B, V, D, F)(t, pe)
    if F < B:
        out = _make_tc_fill(B, V, D, F)(t.reshape(B, 1), _dd_row(D), out)
    return out


# speedup vs baseline: 1.6027x; 1.6027x over previous
import functools

import numpy as np

import jax
import jax.numpy as jnp
from jax import lax
from jax.experimental import pallas as pl
from jax.experimental.pallas import tpu as pltpu
from jax.experimental.pallas import tpu_sc as plsc


@functools.lru_cache(maxsize=None)
def _make_sc_gather(B, V, D, F):
    info = plsc.get_sparse_core_info()
    NC, NS, L = info.num_cores, info.num_subcores, info.num_lanes
    NW = NC * NS
    assert F % NW == 0 and D % L == 0
    b_per_w = F // NW
    CH = 64
    assert b_per_w % CH == 0
    NCH = b_per_w // CH
    NBUF = 3
    LEAD = NBUF - 1
    mesh = plsc.VectorSubcoreMesh(core_axis_name="c", subcore_axis_name="s")

    @functools.partial(
        pl.kernel,
        mesh=mesh,
        out_type=jax.ShapeDtypeStruct((B, D), jnp.float32),
        scratch_types=[
            pltpu.VMEM((b_per_w,), jnp.float32),
            pltpu.VMEM((NCH, CH), jnp.int32),
            pltpu.VMEM((NBUF, CH, D), jnp.float32),
        ]
        + [pltpu.SemaphoreType.DMA] * (2 * NBUF),
    )
    def k(t_hbm, pe_hbm, out_hbm, t_v, idx_v, rows_v, *sems):
        gsem = sems[:NBUF]
        osem = sems[NBUF:]
        wid = lax.axis_index("s") * NC + lax.axis_index("c")
        base = wid * b_per_w
        pltpu.sync_copy(t_hbm.at[pl.ds(base, b_per_w)], t_v)
        scale = jnp.float32(V - 1)

        def compute_idx(c):
            for j in range(CH // L):
                tv = t_v[pl.ds(c * CH + j * L, L)]
                iv = jnp.clip((tv * scale).astype(jnp.int32), 0, V - 1)
                idx_v[c, pl.ds(j * L, L)] = iv

        def gather(c):
            b = c % NBUF
            return pltpu.async_copy(pe_hbm.at[idx_v.at[c]], rows_v.at[b], gsem[b])

        def put(c):
            b = c % NBUF
            return pltpu.async_copy(
                rows_v.at[b], out_hbm.at[pl.ds(base + c * CH, CH)], osem[b])

        gpend = [None] * NBUF
        opend = [None] * NBUF
        for i in range(NCH + LEAD):
            cg = i
            if cg < NCH:
                b = cg % NBUF
                if opend[b] is not None:
                    opend[b].wait()
                    opend[b] = None
                compute_idx(cg)
                gpend[b] = gather(cg)
            cp = i - LEAD
            if 0 <= cp < NCH:
                b = cp % NBUF
                gpend[b].wait()
                opend[b] = put(cp)
        for p in opend:
            if p is not None:
                p.wait()

    return k


_SIN_C = (0.9999997069576652, -0.16666577198095364, 0.00833255799844433,
          -0.00019812572238327207, 2.7040473315129154e-06,
          -2.05340800751852e-08)
_PI2_HI = 6.28125
_PI2_LO = 2.0 * np.pi - _PI2_HI
_INV_2PI = 1.0 / (2.0 * np.pi)


@functools.lru_cache(maxsize=None)
def _make_tc_fill(B, V, D, F, R=512):
    N = B - F
    assert N % R == 0 and F % R == 0
    nblk = N // R
    fblk = F // R
    scale = float(V - 1)

    def body(t_ref, dd_ref, bias_ref, shift_ref, out_ref):
        tv = t_ref[...]
        idx = jnp.clip((tv * scale).astype(jnp.int32), 0, V - 1)
        pos = idx.astype(jnp.float32)
        x = pos * dd_ref[...]
        m = x * _INV_2PI + bias_ref[...]
        k = jnp.floor(m + 0.5)
        r = (x - k * _PI2_HI) - k * _PI2_LO + shift_ref[...]
        r2 = r * r
        p = jnp.float32(_SIN_C[5])
        for c in (_SIN_C[4], _SIN_C[3], _SIN_C[2], _SIN_C[1], _SIN_C[0]):
            p = p * r2 + c
        out_ref[...] = p * r

    return pl.pallas_call(
        body,
        grid=(nblk,),
        in_specs=[
            pl.BlockSpec((R, 1), lambda i: (i + fblk, 0)),
            pl.BlockSpec((1, D), lambda i: (0, 0)),
            pl.BlockSpec((1, D), lambda i: (0, 0)),
            pl.BlockSpec((1, D), lambda i: (0, 0)),
        ],
        out_specs=pl.BlockSpec((R, D), lambda i: (i, 0)),
        out_shape=jax.ShapeDtypeStruct((N, D), jnp.float32),
    )


@functools.lru_cache(maxsize=None)
def _const_rows(D):
    div = np.exp(np.arange(0, D, 2).astype(np.float32) * (-np.log(10000.0) / D))
    dd = np.repeat(div, 2).reshape(1, D)
    bias = np.tile(np.array([0.0, 0.25], dtype=np.float32), D // 2).reshape(1, D)
    shift = np.tile(np.array([0.0, np.pi / 2], dtype=np.float32), D // 2).reshape(1, D)
    return jnp.asarray(dd), jnp.asarray(bias), jnp.asarray(shift)


_SC_ROWS_NUM, _SC_ROWS_DEN = 1, 2


def kernel(t, pe):
    B, = t.shape
    V, D = pe.shape
    F = (B * _SC_ROWS_NUM // _SC_ROWS_DEN) // 2048 * 2048
    out = _make_sc_gather(B, V, D, F)(t, pe)
    if F < B:
        dd, bias, shift = _const_rows(D)
        tail = _make_tc_fill(B, V, D, F)(t.reshape(B, 1), dd, bias, shift)
        out = lax.dynamic_update_slice(out, tail, (F, 0))
    return out


# speedup vs baseline: 1.8900x; 1.1793x over previous
import functools

import numpy as np

import jax
import jax.numpy as jnp
from jax import lax
from jax.experimental import pallas as pl
from jax.experimental.pallas import tpu as pltpu
from jax.experimental.pallas import tpu_sc as plsc


@functools.lru_cache(maxsize=None)
def _make_sc_gather(B, V, D, F):
    info = plsc.get_sparse_core_info()
    NC, NS, L = info.num_cores, info.num_subcores, info.num_lanes
    NW = NC * NS
    assert F % NW == 0 and D % L == 0
    b_per_w = F // NW
    CH = 64
    assert b_per_w % CH == 0
    NCH = b_per_w // CH
    NBUF = 3
    LEAD = NBUF - 1
    mesh = plsc.VectorSubcoreMesh(core_axis_name="c", subcore_axis_name="s")

    @functools.partial(
        pl.kernel,
        mesh=mesh,
        out_type=jax.ShapeDtypeStruct((B, D), jnp.float32),
        scratch_types=[
            pltpu.VMEM((b_per_w,), jnp.float32),
            pltpu.VMEM((NCH, CH), jnp.int32),
            pltpu.VMEM((NBUF, CH, D), jnp.float32),
        ]
        + [pltpu.SemaphoreType.DMA] * (2 * NBUF),
    )
    def k(t_hbm, pe_hbm, out_hbm, t_v, idx_v, rows_v, *sems):
        gsem = sems[:NBUF]
        osem = sems[NBUF:]
        wid = lax.axis_index("s") * NC + lax.axis_index("c")
        base = wid * b_per_w
        pltpu.sync_copy(t_hbm.at[pl.ds(base, b_per_w)], t_v)
        scale = jnp.float32(V - 1)

        def compute_idx(c):
            for j in range(CH // L):
                tv = t_v[pl.ds(c * CH + j * L, L)]
                iv = jnp.clip((tv * scale).astype(jnp.int32), 0, V - 1)
                idx_v[c, pl.ds(j * L, L)] = iv

        def gather(c):
            b = c % NBUF
            return pltpu.async_copy(pe_hbm.at[idx_v.at[c]], rows_v.at[b], gsem[b])

        def put(c):
            b = c % NBUF
            return pltpu.async_copy(
                rows_v.at[b], out_hbm.at[pl.ds(base + c * CH, CH)], osem[b])

        gpend = [None] * NBUF
        opend = [None] * NBUF
        for i in range(NCH + LEAD):
            cg = i
            if cg < NCH:
                b = cg % NBUF
                if opend[b] is not None:
                    opend[b].wait()
                    opend[b] = None
                compute_idx(cg)
                gpend[b] = gather(cg)
            cp = i - LEAD
            if 0 <= cp < NCH:
                b = cp % NBUF
                gpend[b].wait()
                opend[b] = put(cp)
        for p in opend:
            if p is not None:
                p.wait()

    return k


_SIN_C = (0.9999997069576652, -0.16666577198095364, 0.00833255799844433,
          -0.00019812572238327207, 2.7040473315129154e-06,
          -2.05340800751852e-08)
_PI2_HI = 6.28125
_PI2_LO = 2.0 * np.pi - _PI2_HI
_INV_2PI = 1.0 / (2.0 * np.pi)


@functools.lru_cache(maxsize=None)
def _make_tc_fill(B, V, D, F, R=512):
    N = B - F
    assert N % R == 0 and F % R == 0
    nblk = N // R
    fblk = F // R
    scale = float(V - 1)

    def body(t_ref, dd_ref, bias_ref, shift_ref, out_ref):
        tv = jnp.transpose(t_ref[0])
        idx = jnp.clip((tv * scale).astype(jnp.int32), 0, V - 1)
        pos = idx.astype(jnp.float32)
        x = pos * dd_ref[...]
        m = x * _INV_2PI + bias_ref[...]
        k = jnp.floor(m + 0.5)
        r = (x - k * _PI2_HI) - k * _PI2_LO + shift_ref[...]
        r2 = r * r
        p = jnp.float32(_SIN_C[5])
        for c in (_SIN_C[4], _SIN_C[3], _SIN_C[2], _SIN_C[1], _SIN_C[0]):
            p = p * r2 + c
        out_ref[...] = p * r

    return pl.pallas_call(
        body,
        grid=(nblk,),
        in_specs=[
            pl.BlockSpec((1, 1, R), lambda i: (i + fblk, 0, 0)),
            pl.BlockSpec((1, D), lambda i: (0, 0)),
            pl.BlockSpec((1, D), lambda i: (0, 0)),
            pl.BlockSpec((1, D), lambda i: (0, 0)),
        ],
        out_specs=pl.BlockSpec((R, D), lambda i: (i, 0)),
        out_shape=jax.ShapeDtypeStruct((N, D), jnp.float32),
    )


@functools.lru_cache(maxsize=None)
def _const_rows(D):
    div = np.exp(np.arange(0, D, 2).astype(np.float32) * (-np.log(10000.0) / D))
    dd = np.repeat(div, 2).reshape(1, D)
    bias = np.tile(np.array([0.0, 0.25], dtype=np.float32), D // 2).reshape(1, D)
    shift = np.tile(np.array([0.0, np.pi / 2], dtype=np.float32), D // 2).reshape(1, D)
    return jnp.asarray(dd), jnp.asarray(bias), jnp.asarray(shift)


_SC_ROWS_NUM, _SC_ROWS_DEN = 5, 8


def kernel(t, pe):
    B, = t.shape
    V, D = pe.shape
    R = 512
    F = (B * _SC_ROWS_NUM // _SC_ROWS_DEN) // 2048 * 2048
    out = _make_sc_gather(B, V, D, F)(t, pe)
    if F < B:
        dd, bias, shift = _const_rows(D)
        tail = _make_tc_fill(B, V, D, F, R)(t.reshape(B // R, 1, R), dd, bias, shift)
        out = lax.dynamic_update_slice(out, tail, (F, 0))
    return out


# speedup vs baseline: 2.0763x; 1.0986x over previous
import functools

import jax
import jax.numpy as jnp
from jax import lax
from jax.experimental import pallas as pl
from jax.experimental.pallas import tpu as pltpu
from jax.experimental.pallas import tpu_sc as plsc


@functools.lru_cache(maxsize=None)
def _make_pe_gather(B, V, D):
    info = plsc.get_sparse_core_info()
    NC, NS, L = info.num_cores, info.num_subcores, info.num_lanes
    NW = NC * NS
    assert B % NW == 0 and D % L == 0
    b_per_w = B // NW
    CH = 64
    assert b_per_w % CH == 0
    NCH = b_per_w // CH
    NBUF = 3
    LEAD = NBUF - 1
    mesh = plsc.VectorSubcoreMesh(core_axis_name="c", subcore_axis_name="s")

    @functools.partial(
        pl.kernel,
        mesh=mesh,
        out_type=jax.ShapeDtypeStruct((B, D), jnp.float32),
        scratch_types=[
            pltpu.VMEM((b_per_w,), jnp.float32),
            pltpu.VMEM((NCH, CH), jnp.int32),
            pltpu.VMEM((NBUF, CH, D), jnp.float32),
        ]
        + [pltpu.SemaphoreType.DMA] * (2 * NBUF),
    )
    def k(t_hbm, pe_hbm, out_hbm, t_v, idx_v, rows_v, *sems):
        gsem = sems[:NBUF]
        osem = sems[NBUF:]
        wid = lax.axis_index("s") * NC + lax.axis_index("c")
        base = wid * b_per_w
        pltpu.sync_copy(t_hbm.at[pl.ds(base, b_per_w)], t_v)
        scale = jnp.float32(V - 1)

        def compute_idx(c):
            for j in range(CH // L):
                tv = t_v[pl.ds(c * CH + j * L, L)]
                iv = jnp.clip((tv * scale).astype(jnp.int32), 0, V - 1)
                idx_v[c, pl.ds(j * L, L)] = iv

        def gather(c):
            b = c % NBUF
            return pltpu.async_copy(pe_hbm.at[idx_v.at[c]], rows_v.at[b], gsem[b])

        def put(c):
            b = c % NBUF
            return pltpu.async_copy(
                rows_v.at[b], out_hbm.at[pl.ds(base + c * CH, CH)], osem[b])

        gpend = [None] * NBUF
        opend = [None] * NBUF
        for i in range(NCH + LEAD):
            cg = i
            if cg < NCH:
                b = cg % NBUF
                if opend[b] is not None:
                    opend[b].wait()
                    opend[b] = None
                compute_idx(cg)
                gpend[b] = gather(cg)
            cp = i - LEAD
            if 0 <= cp < NCH:
                b = cp % NBUF
                gpend[b].wait()
                opend[b] = put(cp)
        for p in opend:
            if p is not None:
                p.wait()

    return k


def kernel(t, pe):
    B, = t.shape
    V, D = pe.shape
    return _make_pe_gather(B, V, D)(t, pe)


# speedup vs baseline: 2.0960x; 1.0095x over previous
import functools

import jax
import jax.numpy as jnp
from jax import lax
from jax.experimental import pallas as pl
from jax.experimental.pallas import tpu as pltpu
from jax.experimental.pallas import tpu_sc as plsc


@functools.lru_cache(maxsize=None)
def _make_pe_gather(B, V, D):
    info = plsc.get_sparse_core_info()
    NC, NS, L = info.num_cores, info.num_subcores, info.num_lanes
    NW = NC * NS
    assert B % NW == 0 and D % L == 0
    b_per_w = B // NW
    CH = 64
    assert b_per_w % CH == 0
    NCH = b_per_w // CH
    NBUF = 3
    LEAD = NBUF - 1
    mesh = plsc.VectorSubcoreMesh(core_axis_name="c", subcore_axis_name="s")

    @functools.partial(
        pl.kernel,
        mesh=mesh,
        out_type=jax.ShapeDtypeStruct((B, D), jnp.float32),
        scratch_types=[
            pltpu.VMEM((b_per_w,), jnp.float32),
            pltpu.VMEM((NCH, CH), jnp.int32),
            pltpu.VMEM((NBUF, CH, D), jnp.float32),
        ]
        + [pltpu.SemaphoreType.DMA] * (2 * NBUF),
    )
    def k(t_hbm, pe_hbm, out_hbm, t_v, idx_v, rows_v, *sems):
        gsem = sems[:NBUF]
        osem = sems[NBUF:]
        wid = lax.axis_index("s") * NC + lax.axis_index("c")
        base = wid * b_per_w
        scale = jnp.float32(V - 1)

        def compute_idx(c):
            for j in range(CH // L):
                tv = t_v[pl.ds(c * CH + j * L, L)]
                iv = jnp.clip((tv * scale).astype(jnp.int32), 0, V - 1)
                idx_v[c, pl.ds(j * L, L)] = iv

        def gather(c):
            b = c % NBUF
            return pltpu.async_copy(pe_hbm.at[idx_v.at[c]], rows_v.at[b], gsem[b])

        def put(c):
            b = c % NBUF
            return pltpu.async_copy(
                rows_v.at[b], out_hbm.at[pl.ds(base + c * CH, CH)], osem[b])

        gpend = [None] * NBUF
        opend = [None] * NBUF
        pltpu.sync_copy(t_hbm.at[pl.ds(base, CH)], t_v.at[pl.ds(0, CH)])
        compute_idx(0)
        gpend[0] = gather(0)
        pltpu.sync_copy(t_hbm.at[pl.ds(base + CH, b_per_w - CH)],
                        t_v.at[pl.ds(CH, b_per_w - CH)])
        for i in range(1, NCH + LEAD):
            cg = i
            if cg < NCH:
                b = cg % NBUF
                if opend[b] is not None:
                    opend[b].wait()
                    opend[b] = None
                compute_idx(cg)
                gpend[b] = gather(cg)
            cp = i - LEAD
            if 0 <= cp < NCH:
                b = cp % NBUF
                gpend[b].wait()
                opend[b] = put(cp)
        for p in opend:
            if p is not None:
                p.wait()

    return k


def kernel(t, pe):
    B, = t.shape
    V, D = pe.shape
    return _make_pe_gather(B, V, D)(t, pe)
